# Initial kernel scaffold; baseline (speedup 1.0000x reference)
import jax
import jax.numpy as jnp
from jax.experimental import pallas as pl


def kernel(x, edge_index, W1, b1, W2, b2, W3, b3, Wg, a_src, a_dst, bg, Wfc, bfc):
    raise NotImplementedError("write your pallas kernel here")


# speedup vs baseline: 3.5377x; 3.5377x over previous
import functools

import jax
import jax.numpy as jnp
from jax import lax
from jax.experimental import pallas as pl
from jax.experimental.pallas import tpu as pltpu
from jax.experimental.pallas import tpu_sc as plsc

NN = 10000
EE = 160000
WIN = 400
NWIN = NN // WIN
CHUNK = 256
NCH = EE // CHUNK
TPAIR = NCH + (NWIN - 1) + NWIN


def _leaky(v):
    return jnp.where(v >= 0, v, 0.2 * v)


def _sc_gather(table, idx):
    B = idx.shape[0]
    D = table.shape[1]
    NWK = 32
    bpw = B // NWK
    CH = 200
    nchunk = bpw // CH
    mesh = plsc.VectorSubcoreMesh(core_axis_name="c", subcore_axis_name="s")

    @functools.partial(
        pl.kernel, mesh=mesh,
        out_type=jax.ShapeDtypeStruct((B, D), jnp.float32),
        scratch_types=[
            pltpu.VMEM((CH,), jnp.int32),
            pltpu.VMEM((CH, D), jnp.float32),
            pltpu.SemaphoreType.DMA,
        ],
    )
    def k(table_hbm, idx_hbm, out_hbm, idx_v, rows_v, sem):
        wid = lax.axis_index("s") * 2 + lax.axis_index("c")
        base = wid * bpw
        for i in range(nchunk):
            off = base + i * CH
            pltpu.sync_copy(idx_hbm.at[pl.ds(off, CH)], idx_v)
            pltpu.async_copy(table_hbm.at[idx_v], rows_v, sem).wait()
            pltpu.sync_copy(rows_v, out_hbm.at[pl.ds(off, CH)])

    return k(table, idx)


def _build_pairs(dst_s):
    w_lo = dst_s[0::CHUNK] // WIN
    w_hi = dst_s[CHUNK - 1::CHUNK] // WIN
    npairs = w_hi - w_lo + 1
    cum = jnp.concatenate([jnp.zeros((1,), jnp.int32),
                           jnp.cumsum(npairs, dtype=jnp.int32)])
    total = cum[NCH]
    t = jnp.arange(NCH + NWIN - 1, dtype=jnp.int32)
    j = jnp.clip(jnp.searchsorted(cum, t, side="right").astype(jnp.int32) - 1,
                 0, NCH - 1)
    valid_r = (t < total).astype(jnp.int32)
    w_r = jnp.where(valid_r > 0, w_lo[j] + (t - cum[j]), jnp.int32(10 ** 6))
    w_all = jnp.concatenate([w_r, jnp.arange(NWIN, dtype=jnp.int32)])
    c_all = jnp.concatenate([j, jnp.zeros((NWIN,), jnp.int32)])
    v_all = jnp.concatenate([valid_r, jnp.zeros((NWIN,), jnp.int32)])
    order = jnp.argsort(w_all)
    w_all = jnp.minimum(w_all[order], NWIN - 1)
    c_all = c_all[order]
    v_all = v_all[order]
    wid_ext = jnp.concatenate([w_all, -jnp.ones((1,), jnp.int32)])
    return wid_ext, c_all, v_all


def _first_last(wid_ref, t, w):
    first = jnp.logical_or(t == 0, wid_ref[jnp.maximum(t - 1, 0)] != w)
    last = wid_ref[t + 1] != w
    return first, last


def _onehot(dst_ref, w, valid):
    dstc = dst_ref[0, 0, :]
    rows = lax.broadcasted_iota(jnp.int32, (WIN, CHUNK), 0) + w * WIN
    return jnp.logical_and(dstc[None, :] == rows, valid > 0)


def _deg_body(wid_ref, cid_ref, vld_ref, dst_ref, out_ref):
    t = pl.program_id(0)
    w = wid_ref[t]
    first, last = _first_last(wid_ref, t, w)

    @pl.when(first)
    def _():
        out_ref[...] = jnp.zeros_like(out_ref)

    onehot = _onehot(dst_ref, w, vld_ref[t]).astype(jnp.float32)
    out_ref[...] += jnp.sum(onehot, axis=1, keepdims=True)

    @pl.when(last)
    def _():
        out_ref[...] = 1.0 / jnp.sqrt(out_ref[...] + 1.0)


def _degree_inv(dst_r, wid_ext, cid, vld):
    grid_spec = pltpu.PrefetchScalarGridSpec(
        num_scalar_prefetch=3,
        grid=(TPAIR,),
        in_specs=[pl.BlockSpec((1, 1, CHUNK),
                               lambda t, wid, cid, vld: (cid[t], 0, 0))],
        out_specs=pl.BlockSpec((WIN, 1), lambda t, wid, cid, vld: (wid[t], 0)),
    )
    return pl.pallas_call(
        _deg_body, grid_spec=grid_spec,
        out_shape=jax.ShapeDtypeStruct((NN, 1), jnp.float32),
    )(wid_ext, cid, vld, dst_r)


def _mm_scale(x, w, dinv):
    K = x.shape[1]
    D = w.shape[1]

    def body(x_ref, w_ref, d_ref, o_ref):
        o_ref[...] = d_ref[...] * jnp.dot(
            x_ref[...], w_ref[...], preferred_element_type=jnp.float32)

    return pl.pallas_call(
        body,
        grid=(NWIN,),
        in_specs=[pl.BlockSpec((WIN, K), lambda i: (i, 0)),
                  pl.BlockSpec((K, D), lambda i: (0, 0)),
                  pl.BlockSpec((WIN, 1), lambda i: (i, 0))],
        out_specs=pl.BlockSpec((WIN, D), lambda i: (i, 0)),
        out_shape=jax.ShapeDtypeStruct((NN, D), jnp.float32),
    )(x, w, dinv)


def _gcn_reduce(dst_r, msg, ht, dinv, b, wid_ext, cid, vld):
    D = msg.shape[1]

    def body(wid_ref, cid_ref, vld_ref, dst_ref, msg_ref, ht_ref, d_ref,
             b_ref, out_ref):
        t = pl.program_id(0)
        w = wid_ref[t]
        first, last = _first_last(wid_ref, t, w)

        @pl.when(first)
        def _():
            out_ref[...] = jnp.zeros_like(out_ref)

        onehot = _onehot(dst_ref, w, vld_ref[t]).astype(jnp.float32)
        out_ref[...] += jnp.dot(onehot, msg_ref[...],
                                preferred_element_type=jnp.float32)

        @pl.when(last)
        def _():
            out_ref[...] = jnp.maximum(
                d_ref[...] * (out_ref[...] + ht_ref[...]) + b_ref[...], 0.0)

    grid_spec = pltpu.PrefetchScalarGridSpec(
        num_scalar_prefetch=3,
        grid=(TPAIR,),
        in_specs=[
            pl.BlockSpec((1, 1, CHUNK), lambda t, wid, cid, vld: (cid[t], 0, 0)),
            pl.BlockSpec((CHUNK, D), lambda t, wid, cid, vld: (cid[t], 0)),
            pl.BlockSpec((WIN, D), lambda t, wid, cid, vld: (wid[t], 0)),
            pl.BlockSpec((WIN, 1), lambda t, wid, cid, vld: (wid[t], 0)),
            pl.BlockSpec((1, D), lambda t, wid, cid, vld: (0, 0)),
        ],
        out_specs=pl.BlockSpec((WIN, D), lambda t, wid, cid, vld: (wid[t], 0)),
    )
    return pl.pallas_call(
        body, grid_spec=grid_spec,
        out_shape=jax.ShapeDtypeStruct((NN, D), jnp.float32),
    )(wid_ext, cid, vld, dst_r, msg, ht, dinv, b)


def _gat_pre(x, wg, a2):
    def body(x_ref, w_ref, a_ref, o_ref):
        h = jnp.dot(x_ref[...], w_ref[...], preferred_element_type=jnp.float32)
        asd = jnp.dot(h, a_ref[...], preferred_element_type=jnp.float32)
        o_ref[...] = jnp.concatenate(
            [h, asd, jnp.zeros((WIN, 94), jnp.float32)], axis=1)

    K = x.shape[1]
    return pl.pallas_call(
        body,
        grid=(NWIN,),
        in_specs=[pl.BlockSpec((WIN, K), lambda i: (i, 0)),
                  pl.BlockSpec((K, 32), lambda i: (0, 0)),
                  pl.BlockSpec((32, 2), lambda i: (0, 0))],
        out_specs=pl.BlockSpec((WIN, 128), lambda i: (i, 0)),
        out_shape=jax.ShapeDtypeStruct((NN, 128), jnp.float32),
    )(x, wg, a2)


def _edge_logits(dst_ref, msg_ref, tw, w, valid):
    onehot = _onehot(dst_ref, w, valid)
    onef = onehot.astype(jnp.float32)
    ad_w = tw[:, 33:34]
    ad_e = lax.dot_general(ad_w, onef, (((0,), (0,)), ((), ())))
    as_e = msg_ref[...][:, 32:33]
    e_row = _leaky(as_e.reshape(1, CHUNK) + ad_e)
    return onehot, onef, e_row


def _gat_max(dst_r, msgT, T, wid_ext, cid, vld):
    def body(wid_ref, cid_ref, vld_ref, dst_ref, msg_ref, t_ref, out_ref):
        t = pl.program_id(0)
        w = wid_ref[t]
        first, _ = _first_last(wid_ref, t, w)
        tw = t_ref[...]

        @pl.when(first)
        def _():
            out_ref[...] = _leaky(tw[:, 32:33] + tw[:, 33:34])

        onehot, _, e_row = _edge_logits(dst_ref, msg_ref, tw, w, vld_ref[t])
        cm = jnp.max(jnp.where(onehot, e_row, -1e30), axis=1, keepdims=True)
        out_ref[...] = jnp.maximum(out_ref[...], cm)

    grid_spec = pltpu.PrefetchScalarGridSpec(
        num_scalar_prefetch=3,
        grid=(TPAIR,),
        in_specs=[
            pl.BlockSpec((1, 1, CHUNK), lambda t, wid, cid, vld: (cid[t], 0, 0)),
            pl.BlockSpec((CHUNK, 128), lambda t, wid, cid, vld: (cid[t], 0)),
            pl.BlockSpec((WIN, 128), lambda t, wid, cid, vld: (wid[t], 0)),
        ],
        out_specs=pl.BlockSpec((WIN, 1), lambda t, wid, cid, vld: (wid[t], 0)),
    )
    return pl.pallas_call(
        body, grid_spec=grid_spec,
        out_shape=jax.ShapeDtypeStruct((NN, 1), jnp.float32),
    )(wid_ext, cid, vld, dst_r, msgT, T)


def _gat_sum(dst_r, msgT, T, m, bg, wid_ext, cid, vld):
    def body(wid_ref, cid_ref, vld_ref, dst_ref, msg_ref, t_ref, m_ref,
             b_ref, out_ref, s_ref):
        t = pl.program_id(0)
        w = wid_ref[t]
        first, last = _first_last(wid_ref, t, w)
        tw = t_ref[...]
        mw = m_ref[...]

        @pl.when(first)
        def _():
            e_self = _leaky(tw[:, 32:33] + tw[:, 33:34])
            ex_self = jnp.exp(e_self - mw)
            s_ref[...] = ex_self
            out_ref[...] = ex_self * tw[:, :32]

        onehot, onef, e_row = _edge_logits(dst_ref, msg_ref, tw, w, vld_ref[t])
        m_e = lax.dot_general(mw, onef, (((0,), (0,)), ((), ())))
        col_valid = jnp.any(onehot, axis=0, keepdims=True)
        ex_row = jnp.where(col_valid,
                           jnp.exp(jnp.minimum(e_row - m_e, 0.0)), 0.0)
        hs = msg_ref[...][:, :32]
        out_ref[...] += jnp.dot(onef * ex_row, hs,
                                preferred_element_type=jnp.float32)
        s_ref[...] += lax.dot_general(onef, ex_row,
                                      (((1,), (1,)), ((), ())))

        @pl.when(last)
        def _():
            out_ref[...] = jnp.maximum(
                out_ref[...] / (s_ref[...] + 1e-16) + b_ref[...], 0.0)

    grid_spec = pltpu.PrefetchScalarGridSpec(
        num_scalar_prefetch=3,
        grid=(TPAIR,),
        in_specs=[
            pl.BlockSpec((1, 1, CHUNK), lambda t, wid, cid, vld: (cid[t], 0, 0)),
            pl.BlockSpec((CHUNK, 128), lambda t, wid, cid, vld: (cid[t], 0)),
            pl.BlockSpec((WIN, 128), lambda t, wid, cid, vld: (wid[t], 0)),
            pl.BlockSpec((WIN, 1), lambda t, wid, cid, vld: (wid[t], 0)),
            pl.BlockSpec((1, 32), lambda t, wid, cid, vld: (0, 0)),
        ],
        out_specs=pl.BlockSpec((WIN, 32), lambda t, wid, cid, vld: (wid[t], 0)),
        scratch_shapes=[pltpu.VMEM((WIN, 1), jnp.float32)],
    )
    return pl.pallas_call(
        body, grid_spec=grid_spec,
        out_shape=jax.ShapeDtypeStruct((NN, 32), jnp.float32),
    )(wid_ext, cid, vld, dst_r, msgT, T, m, bg)


def _final(h, wfc, bfc):
    def body(h_ref, w_ref, b_ref, o_ref):
        g = jnp.sum(h_ref[...], axis=0, keepdims=True) * (1.0 / NN)
        o_ref[...] = jnp.dot(g, w_ref[...],
                             preferred_element_type=jnp.float32) + b_ref[...]

    return pl.pallas_call(
        body,
        in_specs=[pl.BlockSpec((NN, 32), lambda: (0, 0)),
                  pl.BlockSpec((32, 2), lambda: (0, 0)),
                  pl.BlockSpec((1, 2), lambda: (0, 0))],
        out_specs=pl.BlockSpec((1, 2), lambda: (0, 0)),
        out_shape=jax.ShapeDtypeStruct((1, 2), jnp.float32),
    )(h, wfc, bfc)


def kernel(x, edge_index, W1, b1, W2, b2, W3, b3, Wg, a_src, a_dst, bg,
           Wfc, bfc):
    src = edge_index[0].astype(jnp.int32)
    dst = edge_index[1].astype(jnp.int32)
    order = jnp.argsort(dst)
    dst_s = dst[order]
    src_s = src[order]
    wid_ext, cid, vld = _build_pairs(dst_s)
    dst_r = dst_s.reshape(NCH, 1, CHUNK)

    dinv = _degree_inv(dst_r, wid_ext, cid, vld)

    W3p = jnp.pad(W3, ((0, 0), (0, 96)))
    b3p = jnp.pad(b3, (0, 96))
    Wgp = jnp.pad(Wg, ((0, 96), (0, 0)))

    h = x
    for W, b in ((W1, b1), (W2, b2), (W3p, b3p)):
        ht = _mm_scale(h, W, dinv)
        msg = _sc_gather(ht, src_s)
        h = _gcn_reduce(dst_r, msg, ht, dinv, b.reshape(1, -1),
                        wid_ext, cid, vld)

    T = _gat_pre(h, Wgp, jnp.stack([a_src, a_dst], axis=1))
    msgT = _sc_gather(T, src_s)
    m = _gat_max(dst_r, msgT, T, wid_ext, cid, vld)
    h5 = _gat_sum(dst_r, msgT, T, m, bg.reshape(1, 32), wid_ext, cid, vld)
    return _final(h5, Wfc, bfc.reshape(1, 2))


# speedup vs baseline: 6.2815x; 1.7756x over previous
import functools

import jax
import jax.numpy as jnp
from jax import lax
from jax.experimental import pallas as pl
from jax.experimental.pallas import tpu as pltpu
from jax.experimental.pallas import tpu_sc as plsc

NN = 10000
EE = 160000
WIN = 400
NWIN = NN // WIN
CHUNK = 3200
NCH = EE // CHUNK
TPAIR = NCH + (NWIN - 1) + NWIN


def _leaky(v):
    return jnp.where(v >= 0, v, 0.2 * v)


def _sc_gather(table, idx):
    B = idx.shape[0]
    D = table.shape[1]
    NWK = 32
    bpw = B // NWK
    CH = 200
    nchunk = bpw // CH
    mesh = plsc.VectorSubcoreMesh(core_axis_name="c", subcore_axis_name="s")

    @functools.partial(
        pl.kernel, mesh=mesh,
        out_type=jax.ShapeDtypeStruct((B, D), jnp.float32),
        scratch_types=[
            pltpu.VMEM((CH,), jnp.int32),
            pltpu.VMEM((CH, D), jnp.float32),
            pltpu.SemaphoreType.DMA,
        ],
    )
    def k(table_hbm, idx_hbm, out_hbm, idx_v, rows_v, sem):
        wid = lax.axis_index("s") * 2 + lax.axis_index("c")
        base = wid * bpw
        for i in range(nchunk):
            off = base + i * CH
            pltpu.sync_copy(idx_hbm.at[pl.ds(off, CH)], idx_v)
            pltpu.async_copy(table_hbm.at[idx_v], rows_v, sem).wait()
            pltpu.sync_copy(rows_v, out_hbm.at[pl.ds(off, CH)])

    return k(table, idx)


def _build_pairs(dst_s):
    w_lo = dst_s[0::CHUNK] // WIN
    w_hi = dst_s[CHUNK - 1::CHUNK] // WIN
    npairs = w_hi - w_lo + 1
    cum = jnp.concatenate([jnp.zeros((1,), jnp.int32),
                           jnp.cumsum(npairs, dtype=jnp.int32)])
    total = cum[NCH]
    t = jnp.arange(NCH + NWIN - 1, dtype=jnp.int32)
    j = jnp.clip(jnp.searchsorted(cum, t, side="right").astype(jnp.int32) - 1,
                 0, NCH - 1)
    valid_r = (t < total).astype(jnp.int32)
    w_r = jnp.where(valid_r > 0, w_lo[j] + (t - cum[j]), jnp.int32(10 ** 6))
    w_all = jnp.concatenate([w_r, jnp.arange(NWIN, dtype=jnp.int32)])
    c_all = jnp.concatenate([j, jnp.zeros((NWIN,), jnp.int32)])
    v_all = jnp.concatenate([valid_r, jnp.zeros((NWIN,), jnp.int32)])
    order = jnp.argsort(w_all)
    w_all = jnp.minimum(w_all[order], NWIN - 1)
    c_all = c_all[order]
    v_all = v_all[order]
    wid_ext = jnp.concatenate([w_all, -jnp.ones((1,), jnp.int32)])
    return wid_ext, c_all, v_all


def _first_last(wid_ref, t, w):
    first = jnp.logical_or(t == 0, wid_ref[jnp.maximum(t - 1, 0)] != w)
    last = wid_ref[t + 1] != w
    return first, last


def _onehot(dst_ref, w, valid):
    dstc = dst_ref[0, 0, :]
    rows = lax.broadcasted_iota(jnp.int32, (WIN, CHUNK), 0) + w * WIN
    return jnp.logical_and(dstc[None, :] == rows, valid > 0)


def _deg_body(wid_ref, cid_ref, vld_ref, dst_ref, out_ref):
    t = pl.program_id(0)
    w = wid_ref[t]
    first, last = _first_last(wid_ref, t, w)

    @pl.when(first)
    def _():
        out_ref[...] = jnp.zeros_like(out_ref)

    onehot = _onehot(dst_ref, w, vld_ref[t]).astype(jnp.float32)
    out_ref[...] += jnp.sum(onehot, axis=1, keepdims=True)

    @pl.when(last)
    def _():
        out_ref[...] = 1.0 / jnp.sqrt(out_ref[...] + 1.0)


def _degree_inv(dst_r, wid_ext, cid, vld):
    grid_spec = pltpu.PrefetchScalarGridSpec(
        num_scalar_prefetch=3,
        grid=(TPAIR,),
        in_specs=[pl.BlockSpec((1, 1, CHUNK),
                               lambda t, wid, cid, vld: (cid[t], 0, 0))],
        out_specs=pl.BlockSpec((WIN, 1), lambda t, wid, cid, vld: (wid[t], 0)),
    )
    return pl.pallas_call(
        _deg_body, grid_spec=grid_spec,
        out_shape=jax.ShapeDtypeStruct((NN, 1), jnp.float32),
    )(wid_ext, cid, vld, dst_r)


def _mm_scale(x, w, dinv):
    K = x.shape[1]
    D = w.shape[1]

    def body(x_ref, w_ref, d_ref, o_ref):
        o_ref[...] = d_ref[...] * jnp.dot(
            x_ref[...], w_ref[...], preferred_element_type=jnp.float32)

    return pl.pallas_call(
        body,
        grid=(NWIN,),
        in_specs=[pl.BlockSpec((WIN, K), lambda i: (i, 0)),
                  pl.BlockSpec((K, D), lambda i: (0, 0)),
                  pl.BlockSpec((WIN, 1), lambda i: (i, 0))],
        out_specs=pl.BlockSpec((WIN, D), lambda i: (i, 0)),
        out_shape=jax.ShapeDtypeStruct((NN, D), jnp.float32),
    )(x, w, dinv)


def _gcn_reduce(dst_r, msg, ht, dinv, b, wid_ext, cid, vld):
    D = msg.shape[1]

    def body(wid_ref, cid_ref, vld_ref, dst_ref, msg_ref, ht_ref, d_ref,
             b_ref, out_ref):
        t = pl.program_id(0)
        w = wid_ref[t]
        first, last = _first_last(wid_ref, t, w)

        @pl.when(first)
        def _():
            out_ref[...] = jnp.zeros_like(out_ref)

        onehot = _onehot(dst_ref, w, vld_ref[t]).astype(jnp.float32)
        out_ref[...] += jnp.dot(onehot, msg_ref[...],
                                preferred_element_type=jnp.float32)

        @pl.when(last)
        def _():
            out_ref[...] = jnp.maximum(
                d_ref[...] * (out_ref[...] + ht_ref[...]) + b_ref[...], 0.0)

    grid_spec = pltpu.PrefetchScalarGridSpec(
        num_scalar_prefetch=3,
        grid=(TPAIR,),
        in_specs=[
            pl.BlockSpec((1, 1, CHUNK), lambda t, wid, cid, vld: (cid[t], 0, 0)),
            pl.BlockSpec((CHUNK, D), lambda t, wid, cid, vld: (cid[t], 0)),
            pl.BlockSpec((WIN, D), lambda t, wid, cid, vld: (wid[t], 0)),
            pl.BlockSpec((WIN, 1), lambda t, wid, cid, vld: (wid[t], 0)),
            pl.BlockSpec((1, D), lambda t, wid, cid, vld: (0, 0)),
        ],
        out_specs=pl.BlockSpec((WIN, D), lambda t, wid, cid, vld: (wid[t], 0)),
    )
    return pl.pallas_call(
        body, grid_spec=grid_spec,
        out_shape=jax.ShapeDtypeStruct((NN, D), jnp.float32),
    )(wid_ext, cid, vld, dst_r, msg, ht, dinv, b)


def _gat_pre(x, wg, a2):
    def body(x_ref, w_ref, a_ref, o_ref):
        h = jnp.dot(x_ref[...], w_ref[...], preferred_element_type=jnp.float32)
        asd = jnp.dot(h, a_ref[...], preferred_element_type=jnp.float32)
        o_ref[...] = jnp.concatenate(
            [h, asd, jnp.zeros((WIN, 94), jnp.float32)], axis=1)

    K = x.shape[1]
    return pl.pallas_call(
        body,
        grid=(NWIN,),
        in_specs=[pl.BlockSpec((WIN, K), lambda i: (i, 0)),
                  pl.BlockSpec((K, 32), lambda i: (0, 0)),
                  pl.BlockSpec((32, 2), lambda i: (0, 0))],
        out_specs=pl.BlockSpec((WIN, 128), lambda i: (i, 0)),
        out_shape=jax.ShapeDtypeStruct((NN, 128), jnp.float32),
    )(x, wg, a2)


def _edge_logits(dst_ref, msg_ref, tw, w, valid):
    onehot = _onehot(dst_ref, w, valid)
    onef = onehot.astype(jnp.float32)
    ad_w = tw[:, 33:34]
    ad_e = lax.dot_general(ad_w, onef, (((0,), (0,)), ((), ())))
    as_e = msg_ref[...][:, 32:33]
    e_row = _leaky(as_e.reshape(1, CHUNK) + ad_e)
    return onehot, onef, e_row


def _gat_max(dst_r, msgT, T, wid_ext, cid, vld):
    def body(wid_ref, cid_ref, vld_ref, dst_ref, msg_ref, t_ref, out_ref):
        t = pl.program_id(0)
        w = wid_ref[t]
        first, _ = _first_last(wid_ref, t, w)
        tw = t_ref[...]

        @pl.when(first)
        def _():
            out_ref[...] = _leaky(tw[:, 32:33] + tw[:, 33:34])

        onehot, _, e_row = _edge_logits(dst_ref, msg_ref, tw, w, vld_ref[t])
        cm = jnp.max(jnp.where(onehot, e_row, -1e30), axis=1, keepdims=True)
        out_ref[...] = jnp.maximum(out_ref[...], cm)

    grid_spec = pltpu.PrefetchScalarGridSpec(
        num_scalar_prefetch=3,
        grid=(TPAIR,),
        in_specs=[
            pl.BlockSpec((1, 1, CHUNK), lambda t, wid, cid, vld: (cid[t], 0, 0)),
            pl.BlockSpec((CHUNK, 128), lambda t, wid, cid, vld: (cid[t], 0)),
            pl.BlockSpec((WIN, 128), lambda t, wid, cid, vld: (wid[t], 0)),
        ],
        out_specs=pl.BlockSpec((WIN, 1), lambda t, wid, cid, vld: (wid[t], 0)),
    )
    return pl.pallas_call(
        body, grid_spec=grid_spec,
        out_shape=jax.ShapeDtypeStruct((NN, 1), jnp.float32),
    )(wid_ext, cid, vld, dst_r, msgT, T)


def _gat_sum(dst_r, msgT, T, m, bg, wid_ext, cid, vld):
    def body(wid_ref, cid_ref, vld_ref, dst_ref, msg_ref, t_ref, m_ref,
             b_ref, out_ref, s_ref):
        t = pl.program_id(0)
        w = wid_ref[t]
        first, last = _first_last(wid_ref, t, w)
        tw = t_ref[...]
        mw = m_ref[...]

        @pl.when(first)
        def _():
            e_self = _leaky(tw[:, 32:33] + tw[:, 33:34])
            ex_self = jnp.exp(e_self - mw)
            s_ref[...] = ex_self
            out_ref[...] = ex_self * tw[:, :32]

        onehot, onef, e_row = _edge_logits(dst_ref, msg_ref, tw, w, vld_ref[t])
        m_e = lax.dot_general(mw, onef, (((0,), (0,)), ((), ())))
        col_valid = jnp.any(onehot, axis=0, keepdims=True)
        ex_row = jnp.where(col_valid,
                           jnp.exp(jnp.minimum(e_row - m_e, 0.0)), 0.0)
        hs = msg_ref[...][:, :32]
        out_ref[...] += jnp.dot(onef * ex_row, hs,
                                preferred_element_type=jnp.float32)
        s_ref[...] += lax.dot_general(onef, ex_row,
                                      (((1,), (1,)), ((), ())))

        @pl.when(last)
        def _():
            out_ref[...] = jnp.maximum(
                out_ref[...] / (s_ref[...] + 1e-16) + b_ref[...], 0.0)

    grid_spec = pltpu.PrefetchScalarGridSpec(
        num_scalar_prefetch=3,
        grid=(TPAIR,),
        in_specs=[
            pl.BlockSpec((1, 1, CHUNK), lambda t, wid, cid, vld: (cid[t], 0, 0)),
            pl.BlockSpec((CHUNK, 128), lambda t, wid, cid, vld: (cid[t], 0)),
            pl.BlockSpec((WIN, 128), lambda t, wid, cid, vld: (wid[t], 0)),
            pl.BlockSpec((WIN, 1), lambda t, wid, cid, vld: (wid[t], 0)),
            pl.BlockSpec((1, 32), lambda t, wid, cid, vld: (0, 0)),
        ],
        out_specs=pl.BlockSpec((WIN, 32), lambda t, wid, cid, vld: (wid[t], 0)),
        scratch_shapes=[pltpu.VMEM((WIN, 1), jnp.float32)],
    )
    return pl.pallas_call(
        body, grid_spec=grid_spec,
        out_shape=jax.ShapeDtypeStruct((NN, 32), jnp.float32),
    )(wid_ext, cid, vld, dst_r, msgT, T, m, bg)


def _final(h, wfc, bfc):
    def body(h_ref, w_ref, b_ref, o_ref):
        g = jnp.sum(h_ref[...], axis=0, keepdims=True) * (1.0 / NN)
        o_ref[...] = jnp.dot(g, w_ref[...],
                             preferred_element_type=jnp.float32) + b_ref[...]

    return pl.pallas_call(
        body,
        in_specs=[pl.BlockSpec((NN, 32), lambda: (0, 0)),
                  pl.BlockSpec((32, 2), lambda: (0, 0)),
                  pl.BlockSpec((1, 2), lambda: (0, 0))],
        out_specs=pl.BlockSpec((1, 2), lambda: (0, 0)),
        out_shape=jax.ShapeDtypeStruct((1, 2), jnp.float32),
    )(h, wfc, bfc)


def kernel(x, edge_index, W1, b1, W2, b2, W3, b3, Wg, a_src, a_dst, bg,
           Wfc, bfc):
    src = edge_index[0].astype(jnp.int32)
    dst = edge_index[1].astype(jnp.int32)
    order = jnp.argsort(dst)
    dst_s = dst[order]
    src_s = src[order]
    wid_ext, cid, vld = _build_pairs(dst_s)
    dst_r = dst_s.reshape(NCH, 1, CHUNK)

    dinv = _degree_inv(dst_r, wid_ext, cid, vld)

    W3p = jnp.pad(W3, ((0, 0), (0, 96)))
    b3p = jnp.pad(b3, (0, 96))
    Wgp = jnp.pad(Wg, ((0, 96), (0, 0)))

    h = x
    for W, b in ((W1, b1), (W2, b2), (W3p, b3p)):
        ht = _mm_scale(h, W, dinv)
        msg = _sc_gather(ht, src_s)
        h = _gcn_reduce(dst_r, msg, ht, dinv, b.reshape(1, -1),
                        wid_ext, cid, vld)

    T = _gat_pre(h, Wgp, jnp.stack([a_src, a_dst], axis=1))
    msgT = _sc_gather(T, src_s)
    m = _gat_max(dst_r, msgT, T, wid_ext, cid, vld)
    h5 = _gat_sum(dst_r, msgT, T, m, bg.reshape(1, 32), wid_ext, cid, vld)
    return _final(h5, Wfc, bfc.reshape(1, 2))


# speedup vs baseline: 6.5806x; 1.0476x over previous
import functools

import jax
import jax.numpy as jnp
from jax import lax
from jax.experimental import pallas as pl
from jax.experimental.pallas import tpu as pltpu
from jax.experimental.pallas import tpu_sc as plsc

NN = 10000
EE = 160000
WIN = 400
NWIN = NN // WIN
CHUNK = 3200
NCH = EE // CHUNK
TPAIR = NCH + (NWIN - 1) + NWIN


def _leaky(v):
    return jnp.where(v >= 0, v, 0.2 * v)


def _sc_gather(table, idx):
    B = idx.shape[0]
    D = table.shape[1]
    NWK = 32
    bpw = B // NWK
    CH = min(bpw, ((440 * 1024) // (2 * D * 4)) // 8 * 8)
    nfull = bpw // CH
    rem = bpw - nfull * CH
    chunks = [(i * CH, CH) for i in range(nfull)]
    if rem:
        chunks.append((nfull * CH, rem))
    mesh = plsc.VectorSubcoreMesh(core_axis_name="c", subcore_axis_name="s")

    @functools.partial(
        pl.kernel, mesh=mesh,
        out_type=jax.ShapeDtypeStruct((B, D), jnp.float32),
        scratch_types=[
            pltpu.VMEM((bpw,), jnp.int32),
            pltpu.VMEM((CH, D), jnp.float32),
            pltpu.VMEM((CH, D), jnp.float32),
            pltpu.SemaphoreType.DMA,
            pltpu.SemaphoreType.DMA,
            pltpu.SemaphoreType.DMA,
            pltpu.SemaphoreType.DMA,
        ],
    )
    def k(table_hbm, idx_hbm, out_hbm, idx_v, r0, r1, g0, g1, o0, o1):
        wid = lax.axis_index("s") * 2 + lax.axis_index("c")
        base = wid * bpw
        pltpu.sync_copy(idx_hbm.at[pl.ds(base, bpw)], idx_v)
        rows = (r0, r1)
        gsem = (g0, g1)
        osem = (o0, o1)

        def buf(b, ch):
            return rows[b] if ch == CH else rows[b].at[pl.ds(0, ch)]

        gh = [None, None]
        oh = [None, None]
        for i, (off, ch) in enumerate(chunks):
            b = i & 1
            if oh[b] is not None:
                oh[b].wait()
            gh[b] = pltpu.async_copy(
                table_hbm.at[idx_v.at[pl.ds(off, ch)]], buf(b, ch), gsem[b])
            if i >= 1:
                po, pch = chunks[i - 1]
                gh[1 - b].wait()
                oh[1 - b] = pltpu.async_copy(
                    buf(1 - b, pch), out_hbm.at[pl.ds(base + po, pch)],
                    osem[1 - b])
        lo, lch = chunks[-1]
        lb = (len(chunks) - 1) & 1
        gh[lb].wait()
        oh[lb] = pltpu.async_copy(
            buf(lb, lch), out_hbm.at[pl.ds(base + lo, lch)], osem[lb])
        if len(chunks) >= 2 and oh[1 - lb] is not None:
            oh[1 - lb].wait()
        oh[lb].wait()

    return k(table, idx)


def _build_pairs(dst_s):
    w_lo = dst_s[0::CHUNK] // WIN
    w_hi = dst_s[CHUNK - 1::CHUNK] // WIN
    npairs = w_hi - w_lo + 1
    cum = jnp.concatenate([jnp.zeros((1,), jnp.int32),
                           jnp.cumsum(npairs, dtype=jnp.int32)])
    total = cum[NCH]
    t = jnp.arange(NCH + NWIN - 1, dtype=jnp.int32)
    j = jnp.clip(jnp.searchsorted(cum, t, side="right").astype(jnp.int32) - 1,
                 0, NCH - 1)
    valid_r = (t < total).astype(jnp.int32)
    w_r = jnp.where(valid_r > 0, w_lo[j] + (t - cum[j]), jnp.int32(10 ** 6))
    w_all = jnp.concatenate([w_r, jnp.arange(NWIN, dtype=jnp.int32)])
    c_all = jnp.concatenate([j, jnp.zeros((NWIN,), jnp.int32)])
    v_all = jnp.concatenate([valid_r, jnp.zeros((NWIN,), jnp.int32)])
    order = jnp.argsort(w_all)
    w_all = jnp.minimum(w_all[order], NWIN - 1)
    c_all = c_all[order]
    v_all = v_all[order]
    wid_ext = jnp.concatenate([w_all, -jnp.ones((1,), jnp.int32)])
    return wid_ext, c_all, v_all


def _first_last(wid_ref, t, w):
    first = jnp.logical_or(t == 0, wid_ref[jnp.maximum(t - 1, 0)] != w)
    last = wid_ref[t + 1] != w
    return first, last


def _onehot(dst_ref, w, valid):
    dstc = dst_ref[0, 0, :]
    rows = lax.broadcasted_iota(jnp.int32, (WIN, CHUNK), 0) + w * WIN
    return jnp.logical_and(dstc[None, :] == rows, valid > 0)


def _deg_body(wid_ref, cid_ref, vld_ref, dst_ref, out_ref):
    t = pl.program_id(0)
    w = wid_ref[t]
    first, last = _first_last(wid_ref, t, w)

    @pl.when(first)
    def _():
        out_ref[...] = jnp.zeros_like(out_ref)

    onehot = _onehot(dst_ref, w, vld_ref[t]).astype(jnp.float32)
    out_ref[...] += jnp.sum(onehot, axis=1, keepdims=True)

    @pl.when(last)
    def _():
        out_ref[...] = 1.0 / jnp.sqrt(out_ref[...] + 1.0)


def _degree_inv(dst_r, wid_ext, cid, vld):
    grid_spec = pltpu.PrefetchScalarGridSpec(
        num_scalar_prefetch=3,
        grid=(TPAIR,),
        in_specs=[pl.BlockSpec((1, 1, CHUNK),
                               lambda t, wid, cid, vld: (cid[t], 0, 0))],
        out_specs=pl.BlockSpec((WIN, 1), lambda t, wid, cid, vld: (wid[t], 0)),
    )
    return pl.pallas_call(
        _deg_body, grid_spec=grid_spec,
        out_shape=jax.ShapeDtypeStruct((NN, 1), jnp.float32),
    )(wid_ext, cid, vld, dst_r)


def _mm_scale(x, w, dinv):
    K = x.shape[1]
    D = w.shape[1]

    def body(x_ref, w_ref, d_ref, o_ref):
        o_ref[...] = d_ref[...] * jnp.dot(
            x_ref[...], w_ref[...], preferred_element_type=jnp.float32)

    return pl.pallas_call(
        body,
        grid=(NWIN,),
        in_specs=[pl.BlockSpec((WIN, K), lambda i: (i, 0)),
                  pl.BlockSpec((K, D), lambda i: (0, 0)),
                  pl.BlockSpec((WIN, 1), lambda i: (i, 0))],
        out_specs=pl.BlockSpec((WIN, D), lambda i: (i, 0)),
        out_shape=jax.ShapeDtypeStruct((NN, D), jnp.float32),
    )(x, w, dinv)


def _gcn_reduce(dst_r, msg, ht, dinv, b, wid_ext, cid, vld):
    D = msg.shape[1]

    def body(wid_ref, cid_ref, vld_ref, dst_ref, msg_ref, ht_ref, d_ref,
             b_ref, out_ref):
        t = pl.program_id(0)
        w = wid_ref[t]
        first, last = _first_last(wid_ref, t, w)

        @pl.when(first)
        def _():
            out_ref[...] = jnp.zeros_like(out_ref)

        onehot = _onehot(dst_ref, w, vld_ref[t]).astype(jnp.float32)
        out_ref[...] += jnp.dot(onehot, msg_ref[...],
                                preferred_element_type=jnp.float32)

        @pl.when(last)
        def _():
            out_ref[...] = jnp.maximum(
                d_ref[...] * (out_ref[...] + ht_ref[...]) + b_ref[...], 0.0)

    grid_spec = pltpu.PrefetchScalarGridSpec(
        num_scalar_prefetch=3,
        grid=(TPAIR,),
        in_specs=[
            pl.BlockSpec((1, 1, CHUNK), lambda t, wid, cid, vld: (cid[t], 0, 0)),
            pl.BlockSpec((CHUNK, D), lambda t, wid, cid, vld: (cid[t], 0)),
            pl.BlockSpec((WIN, D), lambda t, wid, cid, vld: (wid[t], 0)),
            pl.BlockSpec((WIN, 1), lambda t, wid, cid, vld: (wid[t], 0)),
            pl.BlockSpec((1, D), lambda t, wid, cid, vld: (0, 0)),
        ],
        out_specs=pl.BlockSpec((WIN, D), lambda t, wid, cid, vld: (wid[t], 0)),
    )
    return pl.pallas_call(
        body, grid_spec=grid_spec,
        out_shape=jax.ShapeDtypeStruct((NN, D), jnp.float32),
    )(wid_ext, cid, vld, dst_r, msg, ht, dinv, b)


def _gat_pre(x, wg, a2):
    def body(x_ref, w_ref, a_ref, o_ref):
        h = jnp.dot(x_ref[...], w_ref[...], preferred_element_type=jnp.float32)
        asd = jnp.dot(h, a_ref[...], preferred_element_type=jnp.float32)
        o_ref[...] = jnp.concatenate(
            [h, asd, jnp.zeros((WIN, 94), jnp.float32)], axis=1)

    K = x.shape[1]
    return pl.pallas_call(
        body,
        grid=(NWIN,),
        in_specs=[pl.BlockSpec((WIN, K), lambda i: (i, 0)),
                  pl.BlockSpec((K, 32), lambda i: (0, 0)),
                  pl.BlockSpec((32, 2), lambda i: (0, 0))],
        out_specs=pl.BlockSpec((WIN, 128), lambda i: (i, 0)),
        out_shape=jax.ShapeDtypeStruct((NN, 128), jnp.float32),
    )(x, wg, a2)


def _edge_logits(dst_ref, msg_ref, tw, w, valid):
    onehot = _onehot(dst_ref, w, valid)
    onef = onehot.astype(jnp.float32)
    ad_w = tw[:, 33:34]
    ad_e = lax.dot_general(ad_w, onef, (((0,), (0,)), ((), ())))
    as_e = msg_ref[...][:, 32:33]
    e_row = _leaky(as_e.reshape(1, CHUNK) + ad_e)
    return onehot, onef, e_row


def _gat_max(dst_r, msgT, T, wid_ext, cid, vld):
    def body(wid_ref, cid_ref, vld_ref, dst_ref, msg_ref, t_ref, out_ref):
        t = pl.program_id(0)
        w = wid_ref[t]
        first, _ = _first_last(wid_ref, t, w)
        tw = t_ref[...]

        @pl.when(first)
        def _():
            out_ref[...] = _leaky(tw[:, 32:33] + tw[:, 33:34])

        onehot, _, e_row = _edge_logits(dst_ref, msg_ref, tw, w, vld_ref[t])
        cm = jnp.max(jnp.where(onehot, e_row, -1e30), axis=1, keepdims=True)
        out_ref[...] = jnp.maximum(out_ref[...], cm)

    grid_spec = pltpu.PrefetchScalarGridSpec(
        num_scalar_prefetch=3,
        grid=(TPAIR,),
        in_specs=[
            pl.BlockSpec((1, 1, CHUNK), lambda t, wid, cid, vld: (cid[t], 0, 0)),
            pl.BlockSpec((CHUNK, 128), lambda t, wid, cid, vld: (cid[t], 0)),
            pl.BlockSpec((WIN, 128), lambda t, wid, cid, vld: (wid[t], 0)),
        ],
        out_specs=pl.BlockSpec((WIN, 1), lambda t, wid, cid, vld: (wid[t], 0)),
    )
    return pl.pallas_call(
        body, grid_spec=grid_spec,
        out_shape=jax.ShapeDtypeStruct((NN, 1), jnp.float32),
    )(wid_ext, cid, vld, dst_r, msgT, T)


def _gat_sum(dst_r, msgT, T, m, bg, wid_ext, cid, vld):
    def body(wid_ref, cid_ref, vld_ref, dst_ref, msg_ref, t_ref, m_ref,
             b_ref, out_ref, s_ref):
        t = pl.program_id(0)
        w = wid_ref[t]
        first, last = _first_last(wid_ref, t, w)
        tw = t_ref[...]
        mw = m_ref[...]

        @pl.when(first)
        def _():
            e_self = _leaky(tw[:, 32:33] + tw[:, 33:34])
            ex_self = jnp.exp(e_self - mw)
            s_ref[...] = ex_self
            out_ref[...] = ex_self * tw[:, :32]

        onehot, onef, e_row = _edge_logits(dst_ref, msg_ref, tw, w, vld_ref[t])
        m_e = lax.dot_general(mw, onef, (((0,), (0,)), ((), ())))
        col_valid = jnp.any(onehot, axis=0, keepdims=True)
        ex_row = jnp.where(col_valid,
                           jnp.exp(jnp.minimum(e_row - m_e, 0.0)), 0.0)
        hs = msg_ref[...][:, :32]
        out_ref[...] += jnp.dot(onef * ex_row, hs,
                                preferred_element_type=jnp.float32)
        s_ref[...] += lax.dot_general(onef, ex_row,
                                      (((1,), (1,)), ((), ())))

        @pl.when(last)
        def _():
            out_ref[...] = jnp.maximum(
                out_ref[...] / (s_ref[...] + 1e-16) + b_ref[...], 0.0)

    grid_spec = pltpu.PrefetchScalarGridSpec(
        num_scalar_prefetch=3,
        grid=(TPAIR,),
        in_specs=[
            pl.BlockSpec((1, 1, CHUNK), lambda t, wid, cid, vld: (cid[t], 0, 0)),
            pl.BlockSpec((CHUNK, 128), lambda t, wid, cid, vld: (cid[t], 0)),
            pl.BlockSpec((WIN, 128), lambda t, wid, cid, vld: (wid[t], 0)),
            pl.BlockSpec((WIN, 1), lambda t, wid, cid, vld: (wid[t], 0)),
            pl.BlockSpec((1, 32), lambda t, wid, cid, vld: (0, 0)),
        ],
        out_specs=pl.BlockSpec((WIN, 32), lambda t, wid, cid, vld: (wid[t], 0)),
        scratch_shapes=[pltpu.VMEM((WIN, 1), jnp.float32)],
    )
    return pl.pallas_call(
        body, grid_spec=grid_spec,
        out_shape=jax.ShapeDtypeStruct((NN, 32), jnp.float32),
    )(wid_ext, cid, vld, dst_r, msgT, T, m, bg)


def _final(h, wfc, bfc):
    def body(h_ref, w_ref, b_ref, o_ref):
        g = jnp.sum(h_ref[...], axis=0, keepdims=True) * (1.0 / NN)
        o_ref[...] = jnp.dot(g, w_ref[...],
                             preferred_element_type=jnp.float32) + b_ref[...]

    return pl.pallas_call(
        body,
        in_specs=[pl.BlockSpec((NN, 32), lambda: (0, 0)),
                  pl.BlockSpec((32, 2), lambda: (0, 0)),
                  pl.BlockSpec((1, 2), lambda: (0, 0))],
        out_specs=pl.BlockSpec((1, 2), lambda: (0, 0)),
        out_shape=jax.ShapeDtypeStruct((1, 2), jnp.float32),
    )(h, wfc, bfc)


def kernel(x, edge_index, W1, b1, W2, b2, W3, b3, Wg, a_src, a_dst, bg,
           Wfc, bfc):
    src = edge_index[0].astype(jnp.int32)
    dst = edge_index[1].astype(jnp.int32)
    order = jnp.argsort(dst)
    dst_s = dst[order]
    src_s = src[order]
    wid_ext, cid, vld = _build_pairs(dst_s)
    dst_r = dst_s.reshape(NCH, 1, CHUNK)

    dinv = _degree_inv(dst_r, wid_ext, cid, vld)

    W3p = jnp.pad(W3, ((0, 0), (0, 96)))
    b3p = jnp.pad(b3, (0, 96))
    Wgp = jnp.pad(Wg, ((0, 96), (0, 0)))

    h = x
    for W, b in ((W1, b1), (W2, b2), (W3p, b3p)):
        ht = _mm_scale(h, W, dinv)
        msg = _sc_gather(ht, src_s)
        h = _gcn_reduce(dst_r, msg, ht, dinv, b.reshape(1, -1),
                        wid_ext, cid, vld)

    T = _gat_pre(h, Wgp, jnp.stack([a_src, a_dst], axis=1))
    msgT = _sc_gather(T, src_s)
    m = _gat_max(dst_r, msgT, T, wid_ext, cid, vld)
    h5 = _gat_sum(dst_r, msgT, T, m, bg.reshape(1, 32), wid_ext, cid, vld)
    return _final(h5, Wfc, bfc.reshape(1, 2))
